# Initial kernel scaffold; baseline (speedup 1.0000x reference)
import jax
import jax.numpy as jnp
from jax.experimental import pallas as pl


def kernel(x, edge_index, W1, b1, Wmu, bmu, Wls, bls):
    raise NotImplementedError("write your pallas kernel here")


# speedup vs baseline: 11.1270x; 11.1270x over previous
import functools

import jax
import jax.numpy as jnp
from jax import lax
from jax.experimental import pallas as pl
from jax.experimental.pallas import tpu as pltpu
from jax.experimental.pallas import tpu_sc as plsc

N = 10000
D_IN = 128
D_HID = 256
NC, NS, L = 2, 16, 16
NW = NC * NS
N_ACC = 10240
RPS = N_ACC // NS
C = 128
E_RAW = 320000 + N
CH_PER_W = -(-E_RAW // (NW * C))
E_PAD = NW * C * CH_PER_W
EPW = C * CH_PER_W
CH_DEG = E_PAD // (NS * C)
X_CH = N // L

_mesh = plsc.VectorSubcoreMesh(
    core_axis_name="c", subcore_axis_name="s", num_cores=NC, num_subcores=NS)


@functools.partial(
    pl.kernel,
    out_type=(jax.ShapeDtypeStruct((N_ACC,), jnp.float32),
              jax.ShapeDtypeStruct((N, D_IN), jnp.float32)),
    mesh=_mesh,
    scratch_types=[
        pltpu.VMEM_SHARED((N_ACC,), jnp.float32),
        pltpu.VMEM_SHARED((N_ACC,), jnp.float32),
        pltpu.VMEM((RPS,), jnp.float32),
        pltpu.VMEM((C,), jnp.float32),
        pltpu.VMEM((C,), jnp.int32),
        pltpu.VMEM((N_ACC,), jnp.float32),
        pltpu.VMEM((L, D_IN), jnp.float32),
        pltpu.VMEM((L, D_IN), jnp.float32),
    ],
)
def _sc_prep(dst_hbm, x_hbm, dinv_out, xp_out,
             deg_sh, dinv_sh, work, ones, idxb, dinv_full, xbuf, xpbuf):
    cid = lax.axis_index("c")
    sid = lax.axis_index("s")

    def fill_zero(i, _):
        work[pl.ds(i * L, L)] = jnp.zeros((L,), jnp.float32)
        return 0
    lax.fori_loop(0, RPS // L, fill_zero, 0)

    def fill_one(i, _):
        ones[pl.ds(i * L, L)] = jnp.ones((L,), jnp.float32)
        return 0
    lax.fori_loop(0, C // L, fill_one, 0)

    pltpu.sync_copy(work, deg_sh.at[pl.ds(sid * RPS, RPS)])
    plsc.subcore_barrier()

    def deg_step(k, _):
        base = (sid * CH_DEG + k) * C
        pltpu.sync_copy(dst_hbm.at[pl.ds(base, C)], idxb)
        pltpu.sync_copy(ones, deg_sh.at[idxb], add=True)
        return 0
    lax.fori_loop(0, CH_DEG, deg_step, 0)
    plsc.subcore_barrier()

    pltpu.sync_copy(deg_sh.at[pl.ds(sid * RPS, RPS)], work)

    def rsq(i, _):
        d = jnp.maximum(work[pl.ds(i * L, L)], 1.0)
        bits = lax.bitcast_convert_type(d, jnp.int32)
        bits = jnp.int32(0x5F3759DF) - lax.shift_right_logical(bits, 1)
        y = lax.bitcast_convert_type(bits, jnp.float32)
        y = y * (1.5 - 0.5 * d * y * y)
        y = y * (1.5 - 0.5 * d * y * y)
        y = y * (1.5 - 0.5 * d * y * y)
        work[pl.ds(i * L, L)] = y
        return 0
    lax.fori_loop(0, RPS // L, rsq, 0)

    pltpu.sync_copy(work, dinv_sh.at[pl.ds(sid * RPS, RPS)])

    @pl.when(cid == 0)
    def _():
        pltpu.sync_copy(work, dinv_out.at[pl.ds(sid * RPS, RPS)])

    plsc.subcore_barrier()
    pltpu.sync_copy(dinv_sh, dinv_full)

    wid = sid * NC + cid

    def x_step(k, _):
        t = k * NW + wid

        @pl.when(t < X_CH)
        def _():
            r0 = t * L
            pltpu.sync_copy(x_hbm.at[pl.ds(r0, L)], xbuf)
            dvec = dinv_full[pl.ds(r0, L)]
            for j in range(L):
                dv = dvec[j]
                for v in range(D_IN // L):
                    xpbuf[j, pl.ds(v * L, L)] = xbuf[j, pl.ds(v * L, L)] * dv
            pltpu.sync_copy(xpbuf, xp_out.at[pl.ds(r0, L)])
        return 0
    lax.fori_loop(0, -(-X_CH // NW), x_step, 0)


def _zero_rows(zrows):
    def zfill(i, _):
        for v in range(D_IN // L):
            zrows[i, pl.ds(v * L, L)] = jnp.zeros((L,), jnp.float32)
        return 0
    lax.fori_loop(0, C, zfill, 0)


def _scatter_pass(src_hbm, dst_hbm, tab_hbm, out_hbm, out_row0,
                  acc, rows, zrows, srcb, dstb, sem, sid, wid):
    def zslice(i, _):
        pltpu.sync_copy(zrows, acc.at[pl.ds(sid * RPS + i * C, C)])
        return 0
    lax.fori_loop(0, RPS // C, zslice, 0)
    plsc.subcore_barrier()

    def step(k, _):
        base = wid * EPW + k * C
        pltpu.sync_copy(src_hbm.at[pl.ds(base, C)], srcb)
        pltpu.sync_copy(dst_hbm.at[pl.ds(base, C)], dstb)
        pltpu.async_copy(tab_hbm.at[srcb], rows, sem).wait()
        pltpu.sync_copy(rows, acc.at[dstb], add=True)
        return 0
    lax.fori_loop(0, CH_PER_W, step, 0)
    plsc.subcore_barrier()

    pltpu.sync_copy(acc.at[pl.ds(sid * RPS, RPS)],
                    out_hbm.at[pl.ds(out_row0 + sid * RPS, RPS)])
    plsc.subcore_barrier()


_SC_SCRATCH = [
    pltpu.VMEM_SHARED((N_ACC, D_IN), jnp.float32),
    pltpu.VMEM((C, D_IN), jnp.float32),
    pltpu.VMEM((C, D_IN), jnp.float32),
    pltpu.VMEM((C,), jnp.int32),
    pltpu.VMEM((C,), jnp.int32),
    pltpu.SemaphoreType.DMA,
]


@functools.partial(
    pl.kernel,
    out_type=jax.ShapeDtypeStruct((NC * N_ACC, D_IN), jnp.float32),
    mesh=_mesh, scratch_types=_SC_SCRATCH,
)
def _sc_scatter_x(src_hbm, dst_hbm, tab_hbm, out_hbm,
                  acc, rows, zrows, srcb, dstb, sem):
    cid = lax.axis_index("c")
    sid = lax.axis_index("s")
    wid = sid * NC + cid
    _zero_rows(zrows)
    _scatter_pass(src_hbm, dst_hbm, tab_hbm, out_hbm, cid * N_ACC,
                  acc, rows, zrows, srcb, dstb, sem, sid, wid)


@functools.partial(
    pl.kernel,
    out_type=jax.ShapeDtypeStruct((NC * 2 * N_ACC, D_IN), jnp.float32),
    mesh=_mesh, scratch_types=_SC_SCRATCH,
)
def _sc_scatter_h(src_hbm, dst_hbm, ta_hbm, tb_hbm, out_hbm,
                  acc, rows, zrows, srcb, dstb, sem):
    cid = lax.axis_index("c")
    sid = lax.axis_index("s")
    wid = sid * NC + cid
    _zero_rows(zrows)
    for half, tab in enumerate((ta_hbm, tb_hbm)):
        _scatter_pass(src_hbm, dst_hbm, tab, out_hbm,
                      (cid * 2 + half) * N_ACC,
                      acc, rows, zrows, srcb, dstb, sem, sid, wid)


M_BLK = 400
GRID_M = N // M_BLK


def _tc1_body(gpa, gpb, dv, w1, b1, hpa, hpb):
    d = dv[...]
    g0 = (gpa[...] + gpb[...]) * d
    h = jnp.dot(g0, w1[...], preferred_element_type=jnp.float32) + b1[...]
    h = jnp.maximum(h, 0.0) * d
    hpa[...] = h[:, :D_IN]
    hpb[...] = h[:, D_IN:]


_tc1 = pl.pallas_call(
    _tc1_body,
    grid=(GRID_M,),
    in_specs=[
        pl.BlockSpec((M_BLK, D_IN), lambda i: (i, 0)),
        pl.BlockSpec((M_BLK, D_IN), lambda i: (i, 0)),
        pl.BlockSpec((M_BLK, 1), lambda i: (i, 0)),
        pl.BlockSpec((D_IN, D_HID), lambda i: (0, 0)),
        pl.BlockSpec((1, D_HID), lambda i: (0, 0)),
    ],
    out_specs=[pl.BlockSpec((M_BLK, D_IN), lambda i: (i, 0))] * 2,
    out_shape=[jax.ShapeDtypeStruct((N, D_IN), jnp.float32)] * 2,
)


def _tc2_body(a0, a1, b0, b1, dv, wma, wmb, wla, wlb, bm, bl, mu, ls):
    d = dv[...]
    ga = (a0[...] + a1[...]) * d
    gb = (b0[...] + b1[...]) * d
    mu[...] = (jnp.dot(ga, wma[...], preferred_element_type=jnp.float32)
               + jnp.dot(gb, wmb[...], preferred_element_type=jnp.float32)
               + bm[...])
    ls[...] = (jnp.dot(ga, wla[...], preferred_element_type=jnp.float32)
               + jnp.dot(gb, wlb[...], preferred_element_type=jnp.float32)
               + bl[...])


_tc2 = pl.pallas_call(
    _tc2_body,
    grid=(GRID_M,),
    in_specs=(
        [pl.BlockSpec((M_BLK, D_IN), lambda i: (i, 0))] * 4
        + [pl.BlockSpec((M_BLK, 1), lambda i: (i, 0))]
        + [pl.BlockSpec((D_IN, D_IN), lambda i: (0, 0))] * 4
        + [pl.BlockSpec((1, D_IN), lambda i: (0, 0))] * 2
    ),
    out_specs=[pl.BlockSpec((M_BLK, D_IN), lambda i: (i, 0))] * 2,
    out_shape=[jax.ShapeDtypeStruct((N, D_IN), jnp.float32)] * 2,
)


def kernel(x, edge_index, W1, b1, Wmu, bmu, Wls, bls):
    ei = edge_index.astype(jnp.int32)
    loop = jnp.arange(N, dtype=jnp.int32)
    src = jnp.concatenate(
        [ei[0], loop, jnp.zeros((E_PAD - E_RAW,), jnp.int32)])
    dst = jnp.concatenate(
        [ei[1], loop, jnp.full((E_PAD - E_RAW,), N, jnp.int32)])

    dinv, xp = _sc_prep(dst, x)
    dv = dinv[:N, None]

    s0 = _sc_scatter_x(src, dst, xp)
    hpa, hpb = _tc1(s0[:N], s0[N_ACC:N_ACC + N], dv, W1, b1.reshape(1, -1))

    s1 = _sc_scatter_h(src, dst, hpa, hpb)
    a0 = s1[:N]
    b0 = s1[N_ACC:N_ACC + N]
    a1 = s1[2 * N_ACC:2 * N_ACC + N]
    b1h = s1[3 * N_ACC:3 * N_ACC + N]

    mu, ls = _tc2(a0, a1, b0, b1h, dv,
                  Wmu[:D_IN], Wmu[D_IN:], Wls[:D_IN], Wls[D_IN:],
                  bmu.reshape(1, -1), bls.reshape(1, -1))
    return (mu, ls)


# speedup vs baseline: 14.9773x; 1.3460x over previous
import functools

import jax
import jax.numpy as jnp
from jax import lax
from jax.experimental import pallas as pl
from jax.experimental.pallas import tpu as pltpu
from jax.experimental.pallas import tpu_sc as plsc

N = 10000
D_IN = 128
D_HID = 256
NC, NS, L = 2, 16, 16
NW = NC * NS
N_ACC = 10240
RPS = N_ACC // NS
C = 64
CDEG = 128
E_RAW = 320000 + N
CH_PER_W = -(-E_RAW // (NW * C))
E_PAD = NW * C * CH_PER_W
EPW = C * CH_PER_W
CH_DEG = E_PAD // (NS * CDEG)
X_CH = N // L

_mesh = plsc.VectorSubcoreMesh(
    core_axis_name="c", subcore_axis_name="s", num_cores=NC, num_subcores=NS)


@functools.partial(
    pl.kernel,
    out_type=(jax.ShapeDtypeStruct((N_ACC,), jnp.float32),
              jax.ShapeDtypeStruct((N, D_IN), jnp.float32)),
    mesh=_mesh,
    scratch_types=[
        pltpu.VMEM_SHARED((N_ACC,), jnp.float32),
        pltpu.VMEM_SHARED((N_ACC,), jnp.float32),
        pltpu.VMEM((RPS,), jnp.float32),
        pltpu.VMEM((CDEG,), jnp.float32),
        pltpu.VMEM((CDEG,), jnp.int32),
        pltpu.VMEM((CDEG,), jnp.int32),
        pltpu.VMEM((N_ACC,), jnp.float32),
        pltpu.VMEM((L, D_IN), jnp.float32),
        pltpu.VMEM((L, D_IN), jnp.float32),
        pltpu.SemaphoreType.DMA,
        pltpu.SemaphoreType.DMA,
    ],
)
def _sc_prep(dst_hbm, x_hbm, dinv_out, xp_out,
             deg_sh, dinv_sh, work, ones, idxb0, idxb1, dinv_full,
             xbuf, xpbuf, dsem0, dsem1):
    cid = lax.axis_index("c")
    sid = lax.axis_index("s")

    def fill_zero(i, _):
        work[pl.ds(i * L, L)] = jnp.zeros((L,), jnp.float32)
        return 0
    lax.fori_loop(0, RPS // L, fill_zero, 0)

    def fill_one(i, _):
        ones[pl.ds(i * L, L)] = jnp.ones((L,), jnp.float32)
        return 0
    lax.fori_loop(0, CDEG // L, fill_one, 0)

    pltpu.sync_copy(work, deg_sh.at[pl.ds(sid * RPS, RPS)])
    plsc.subcore_barrier()

    idxb = (idxb0, idxb1)
    dsem = (dsem0, dsem1)
    dbase = sid * CH_DEG * CDEG

    def dfront(k, b, wait_s):
        if wait_s:
            pltpu.make_async_copy(ones, deg_sh.at[idxb[b]], dsem[b]).wait()
        pltpu.sync_copy(dst_hbm.at[pl.ds(dbase + k * CDEG, CDEG)], idxb[b])
        pltpu.async_copy(ones, deg_sh.at[idxb[b]], dsem[b], add=True)

    dfront(0, 0, False)
    dfront(1, 1, False)

    def dgrp(kk, _):
        for b in range(2):
            dfront(2 * kk + b, b, True)
        return 0
    lax.fori_loop(1, CH_DEG // 2, dgrp, 0)
    for b in range(2):
        pltpu.make_async_copy(ones, deg_sh.at[idxb[b]], dsem[b]).wait()
    plsc.subcore_barrier()

    pltpu.sync_copy(deg_sh.at[pl.ds(sid * RPS, RPS)], work)

    def rsq(i, _):
        d = jnp.maximum(work[pl.ds(i * L, L)], 1.0)
        bits = lax.bitcast_convert_type(d, jnp.int32)
        bits = jnp.int32(0x5F3759DF) - lax.shift_right_logical(bits, 1)
        y = lax.bitcast_convert_type(bits, jnp.float32)
        y = y * (1.5 - 0.5 * d * y * y)
        y = y * (1.5 - 0.5 * d * y * y)
        y = y * (1.5 - 0.5 * d * y * y)
        work[pl.ds(i * L, L)] = y
        return 0
    lax.fori_loop(0, RPS // L, rsq, 0)

    pltpu.sync_copy(work, dinv_sh.at[pl.ds(sid * RPS, RPS)])

    @pl.when(cid == 0)
    def _():
        pltpu.sync_copy(work, dinv_out.at[pl.ds(sid * RPS, RPS)])

    plsc.subcore_barrier()
    pltpu.sync_copy(dinv_sh, dinv_full)

    wid = sid * NC + cid

    def x_step(k, _):
        t = k * NW + wid

        @pl.when(t < X_CH)
        def _():
            r0 = t * L
            pltpu.sync_copy(x_hbm.at[pl.ds(r0, L)], xbuf)
            dvec = dinv_full[pl.ds(r0, L)]
            for j in range(L):
                dv = dvec[j]
                for v in range(D_IN // L):
                    xpbuf[j, pl.ds(v * L, L)] = xbuf[j, pl.ds(v * L, L)] * dv
            pltpu.sync_copy(xpbuf, xp_out.at[pl.ds(r0, L)])
        return 0
    lax.fori_loop(0, -(-X_CH // NW), x_step, 0)


def _zero_rows(zrows):
    def zfill(i, _):
        for v in range(D_IN // L):
            zrows[i, pl.ds(v * L, L)] = jnp.zeros((L,), jnp.float32)
        return 0
    lax.fori_loop(0, C, zfill, 0)


NBUF = 4
_GRP0 = 2
_GRPN = (CH_PER_W - 1 - 2 - _GRP0) // NBUF
_EPI0 = _GRP0 + _GRPN * NBUF


def _scatter_pass(src_hbm, dst_hbm, tab_hbm, out_hbm, out_row0,
                  acc, zrows, srcb, dstb, rows, gsem, ssem, sid, wid):
    def zslice(i, _):
        pltpu.sync_copy(zrows, acc.at[pl.ds(sid * RPS + i * C, C)])
        return 0
    lax.fori_loop(0, RPS // C, zslice, 0)
    plsc.subcore_barrier()

    ebase = wid * EPW

    def front(k, b, wait_s):
        if wait_s:
            pltpu.make_async_copy(rows[b], acc.at[dstb[b]], ssem[b]).wait()
        pltpu.sync_copy(src_hbm.at[pl.ds(ebase + k * C, C)], srcb[b])
        pltpu.sync_copy(dst_hbm.at[pl.ds(ebase + k * C, C)], dstb[b])
        pltpu.async_copy(tab_hbm.at[srcb[b]], rows[b], gsem[b])

    def back(k, b):
        del k
        pltpu.make_async_copy(tab_hbm.at[srcb[b]], rows[b], gsem[b]).wait()
        pltpu.async_copy(rows[b], acc.at[dstb[b]], ssem[b], add=True)

    front(0, 0, False)
    front(1, 1, False)
    front(2, 2, False)
    back(0, 0)
    front(3, 3, False)
    back(1, 1)

    def grp(kk, _):
        k0 = _GRP0 + kk * NBUF
        for b4 in range(NBUF):
            k = k0 + b4
            b = (_GRP0 + b4) % NBUF
            front(k + 2, (b + 2) % NBUF, True)
            back(k, b)
        return 0
    lax.fori_loop(0, _GRPN, grp, 0)

    for k in range(_EPI0, CH_PER_W):
        if k + 2 < CH_PER_W:
            front(k + 2, (k + 2) % NBUF, True)
        back(k, k % NBUF)
    for k in range(CH_PER_W - NBUF, CH_PER_W):
        b = k % NBUF
        pltpu.make_async_copy(rows[b], acc.at[dstb[b]], ssem[b]).wait()
    plsc.subcore_barrier()

    pltpu.sync_copy(acc.at[pl.ds(sid * RPS, RPS)],
                    out_hbm.at[pl.ds(out_row0 + sid * RPS, RPS)])
    plsc.subcore_barrier()


_SC_SCRATCH = (
    [pltpu.VMEM_SHARED((N_ACC, D_IN), jnp.float32)]
    + [pltpu.VMEM((C, D_IN), jnp.float32)]
    + [pltpu.VMEM((C,), jnp.int32)] * NBUF
    + [pltpu.VMEM((C,), jnp.int32)] * NBUF
    + [pltpu.VMEM((C, D_IN), jnp.float32)] * NBUF
    + [pltpu.SemaphoreType.DMA] * (2 * NBUF)
)


def _unpack_scratch(scratch):
    acc, zrows = scratch[0], scratch[1]
    srcb = scratch[2:2 + NBUF]
    dstb = scratch[2 + NBUF:2 + 2 * NBUF]
    rows = scratch[2 + 2 * NBUF:2 + 3 * NBUF]
    gsem = scratch[2 + 3 * NBUF:2 + 4 * NBUF]
    ssem = scratch[2 + 4 * NBUF:2 + 5 * NBUF]
    return acc, zrows, srcb, dstb, rows, gsem, ssem


@functools.partial(
    pl.kernel,
    out_type=jax.ShapeDtypeStruct((NC * N_ACC, D_IN), jnp.float32),
    mesh=_mesh, scratch_types=_SC_SCRATCH,
)
def _sc_scatter_x(src_hbm, dst_hbm, tab_hbm, out_hbm, *scratch):
    cid = lax.axis_index("c")
    sid = lax.axis_index("s")
    wid = sid * NC + cid
    acc, zrows, srcb, dstb, rows, gsem, ssem = _unpack_scratch(scratch)
    _zero_rows(zrows)
    _scatter_pass(src_hbm, dst_hbm, tab_hbm, out_hbm, cid * N_ACC,
                  acc, zrows, srcb, dstb, rows, gsem, ssem, sid, wid)


@functools.partial(
    pl.kernel,
    out_type=jax.ShapeDtypeStruct((NC * 2 * N_ACC, D_IN), jnp.float32),
    mesh=_mesh, scratch_types=_SC_SCRATCH,
)
def _sc_scatter_h(src_hbm, dst_hbm, ta_hbm, tb_hbm, out_hbm, *scratch):
    cid = lax.axis_index("c")
    sid = lax.axis_index("s")
    wid = sid * NC + cid
    acc, zrows, srcb, dstb, rows, gsem, ssem = _unpack_scratch(scratch)
    _zero_rows(zrows)
    for half, tab in enumerate((ta_hbm, tb_hbm)):
        _scatter_pass(src_hbm, dst_hbm, tab, out_hbm,
                      (cid * 2 + half) * N_ACC,
                      acc, zrows, srcb, dstb, rows, gsem, ssem, sid, wid)


M_BLK = 400
GRID_M = N // M_BLK


def _tc1_body(gpa, gpb, dv, w1, b1, hpa, hpb):
    d = dv[...]
    g0 = (gpa[...] + gpb[...]) * d
    h = jnp.dot(g0, w1[...], preferred_element_type=jnp.float32) + b1[...]
    h = jnp.maximum(h, 0.0) * d
    hpa[...] = h[:, :D_IN]
    hpb[...] = h[:, D_IN:]


_tc1 = pl.pallas_call(
    _tc1_body,
    grid=(GRID_M,),
    in_specs=[
        pl.BlockSpec((M_BLK, D_IN), lambda i: (i, 0)),
        pl.BlockSpec((M_BLK, D_IN), lambda i: (i, 0)),
        pl.BlockSpec((M_BLK, 1), lambda i: (i, 0)),
        pl.BlockSpec((D_IN, D_HID), lambda i: (0, 0)),
        pl.BlockSpec((1, D_HID), lambda i: (0, 0)),
    ],
    out_specs=[pl.BlockSpec((M_BLK, D_IN), lambda i: (i, 0))] * 2,
    out_shape=[jax.ShapeDtypeStruct((N, D_IN), jnp.float32)] * 2,
)


def _tc2_body(a0, a1, b0, b1, dv, wma, wmb, wla, wlb, bm, bl, mu, ls):
    d = dv[...]
    ga = (a0[...] + a1[...]) * d
    gb = (b0[...] + b1[...]) * d
    mu[...] = (jnp.dot(ga, wma[...], preferred_element_type=jnp.float32)
               + jnp.dot(gb, wmb[...], preferred_element_type=jnp.float32)
               + bm[...])
    ls[...] = (jnp.dot(ga, wla[...], preferred_element_type=jnp.float32)
               + jnp.dot(gb, wlb[...], preferred_element_type=jnp.float32)
               + bl[...])


_tc2 = pl.pallas_call(
    _tc2_body,
    grid=(GRID_M,),
    in_specs=(
        [pl.BlockSpec((M_BLK, D_IN), lambda i: (i, 0))] * 4
        + [pl.BlockSpec((M_BLK, 1), lambda i: (i, 0))]
        + [pl.BlockSpec((D_IN, D_IN), lambda i: (0, 0))] * 4
        + [pl.BlockSpec((1, D_IN), lambda i: (0, 0))] * 2
    ),
    out_specs=[pl.BlockSpec((M_BLK, D_IN), lambda i: (i, 0))] * 2,
    out_shape=[jax.ShapeDtypeStruct((N, D_IN), jnp.float32)] * 2,
)


def kernel(x, edge_index, W1, b1, Wmu, bmu, Wls, bls):
    ei = edge_index.astype(jnp.int32)
    loop = jnp.arange(N, dtype=jnp.int32)
    src = jnp.concatenate(
        [ei[0], loop, jnp.zeros((E_PAD - E_RAW,), jnp.int32)])
    dst = jnp.concatenate(
        [ei[1], loop, jnp.full((E_PAD - E_RAW,), N, jnp.int32)])

    dinv, xp = _sc_prep(dst, x)
    dv = dinv[:N, None]

    s0 = _sc_scatter_x(src, dst, xp)
    hpa, hpb = _tc1(s0[:N], s0[N_ACC:N_ACC + N], dv, W1, b1.reshape(1, -1))

    s1 = _sc_scatter_h(src, dst, hpa, hpb)
    a0 = s1[:N]
    b0 = s1[N_ACC:N_ACC + N]
    a1 = s1[2 * N_ACC:2 * N_ACC + N]
    b1h = s1[3 * N_ACC:3 * N_ACC + N]

    mu, ls = _tc2(a0, a1, b0, b1h, dv,
                  Wmu[:D_IN], Wmu[D_IN:], Wls[:D_IN], Wls[D_IN:],
                  bmu.reshape(1, -1), bls.reshape(1, -1))
    return (mu, ls)


# speedup vs baseline: 16.8454x; 1.1247x over previous
import functools

import jax
import jax.numpy as jnp
from jax import lax
from jax.experimental import pallas as pl
from jax.experimental.pallas import tpu as pltpu
from jax.experimental.pallas import tpu_sc as plsc

N = 10000
D_IN = 128
D_HID = 256
NC, NS, L = 2, 16, 16
NW = NC * NS
N_ACC = 10240
RPS = N_ACC // NS
C = 64
CDEG = 128
E_RAW = 320000 + N
CH_PER_W = -(-E_RAW // (NW * C))
CH_BLK = CH_PER_W // 6
E_PAD = NW * C * CH_PER_W
EPW = C * CH_PER_W
CH_DEG = E_PAD // (NS * CDEG)
X_CH = N // L

_mesh = plsc.VectorSubcoreMesh(
    core_axis_name="c", subcore_axis_name="s", num_cores=NC, num_subcores=NS)


@functools.partial(
    pl.kernel,
    out_type=(jax.ShapeDtypeStruct((N_ACC,), jnp.float32),
              jax.ShapeDtypeStruct((N, D_IN), jnp.float32)),
    mesh=_mesh,
    scratch_types=[
        pltpu.VMEM_SHARED((N_ACC,), jnp.float32),
        pltpu.VMEM_SHARED((N_ACC,), jnp.float32),
        pltpu.VMEM((RPS,), jnp.float32),
        pltpu.VMEM((CDEG,), jnp.float32),
        pltpu.VMEM((CDEG,), jnp.int32),
        pltpu.VMEM((CDEG,), jnp.int32),
        pltpu.VMEM((N_ACC,), jnp.float32),
        pltpu.VMEM((L, D_IN), jnp.float32),
        pltpu.VMEM((L, D_IN), jnp.float32),
        pltpu.SemaphoreType.DMA,
        pltpu.SemaphoreType.DMA,
    ],
)
def _sc_prep(dst_hbm, x_hbm, dinv_out, xp_out,
             deg_sh, dinv_sh, work, ones, idxb0, idxb1, dinv_full,
             xbuf, xpbuf, dsem0, dsem1):
    cid = lax.axis_index("c")
    sid = lax.axis_index("s")

    def fill_zero(i, _):
        work[pl.ds(i * L, L)] = jnp.zeros((L,), jnp.float32)
        return 0
    lax.fori_loop(0, RPS // L, fill_zero, 0)

    def fill_one(i, _):
        ones[pl.ds(i * L, L)] = jnp.ones((L,), jnp.float32)
        return 0
    lax.fori_loop(0, CDEG // L, fill_one, 0)

    pltpu.sync_copy(work, deg_sh.at[pl.ds(sid * RPS, RPS)])
    plsc.subcore_barrier()

    idxb = (idxb0, idxb1)
    dsem = (dsem0, dsem1)
    dbase = sid * CH_DEG * CDEG

    def dfront(k, b, wait_s):
        if wait_s:
            pltpu.make_async_copy(ones, deg_sh.at[idxb[b]], dsem[b]).wait()
        pltpu.sync_copy(dst_hbm.at[pl.ds(dbase + k * CDEG, CDEG)], idxb[b])
        pltpu.async_copy(ones, deg_sh.at[idxb[b]], dsem[b], add=True)

    dfront(0, 0, False)
    dfront(1, 1, False)

    def dgrp(kk, _):
        for b in range(2):
            dfront(2 * kk + b, b, True)
        return 0
    lax.fori_loop(1, CH_DEG // 2, dgrp, 0)
    for b in range(2):
        pltpu.make_async_copy(ones, deg_sh.at[idxb[b]], dsem[b]).wait()
    plsc.subcore_barrier()

    pltpu.sync_copy(deg_sh.at[pl.ds(sid * RPS, RPS)], work)

    def rsq(i, _):
        d = jnp.maximum(work[pl.ds(i * L, L)], 1.0)
        bits = lax.bitcast_convert_type(d, jnp.int32)
        bits = jnp.int32(0x5F3759DF) - lax.shift_right_logical(bits, 1)
        y = lax.bitcast_convert_type(bits, jnp.float32)
        y = y * (1.5 - 0.5 * d * y * y)
        y = y * (1.5 - 0.5 * d * y * y)
        y = y * (1.5 - 0.5 * d * y * y)
        work[pl.ds(i * L, L)] = y
        return 0
    lax.fori_loop(0, RPS // L, rsq, 0)

    pltpu.sync_copy(work, dinv_sh.at[pl.ds(sid * RPS, RPS)])

    @pl.when(cid == 0)
    def _():
        pltpu.sync_copy(work, dinv_out.at[pl.ds(sid * RPS, RPS)])

    plsc.subcore_barrier()
    pltpu.sync_copy(dinv_sh, dinv_full)

    wid = sid * NC + cid

    def x_step(k, _):
        t = k * NW + wid

        @pl.when(t < X_CH)
        def _():
            r0 = t * L
            pltpu.sync_copy(x_hbm.at[pl.ds(r0, L)], xbuf)
            dvec = dinv_full[pl.ds(r0, L)]
            for j in range(L):
                dv = dvec[j]
                for v in range(D_IN // L):
                    xpbuf[j, pl.ds(v * L, L)] = xbuf[j, pl.ds(v * L, L)] * dv
            pltpu.sync_copy(xpbuf, xp_out.at[pl.ds(r0, L)])
        return 0
    lax.fori_loop(0, -(-X_CH // NW), x_step, 0)


NBUF = 4
_GRP0 = 2
_GRPN = (CH_BLK - 1 - 2 - _GRP0) // NBUF
_EPI0 = _GRP0 + _GRPN * NBUF


def _scatter_pass(src2_hbm, dst2_hbm, tab_hbm, out_hbm, out_row0,
                  acc, srcblk, dstblk, zrows, rows, gsem, ssem, sid, wid):
    def zfill(i, _):
        for v in range(D_IN // L):
            zrows[i, pl.ds(v * L, L)] = jnp.zeros((L,), jnp.float32)
        return 0
    lax.fori_loop(0, L, zfill, 0)

    def zslice(i, _):
        pltpu.sync_copy(zrows, acc.at[pl.ds(sid * RPS + i * L, L)])
        return 0
    lax.fori_loop(0, RPS // L, zslice, 0)
    plsc.subcore_barrier()

    def front(k, b, wait_s):
        if wait_s:
            pltpu.make_async_copy(rows[b], acc.at[dstblk.at[0]],
                                  ssem[b]).wait()
        pltpu.async_copy(tab_hbm.at[srcblk.at[k]], rows[b], gsem[b])

    def back(k, b):
        pltpu.make_async_copy(tab_hbm.at[srcblk.at[k]], rows[b],
                              gsem[b]).wait()
        pltpu.async_copy(rows[b], acc.at[dstblk.at[k]], ssem[b], add=True)

    for blk in range(CH_PER_W // CH_BLK):
        wblk = wid * (CH_PER_W // CH_BLK) + blk
        pltpu.sync_copy(src2_hbm.at[wblk], srcblk)
        pltpu.sync_copy(dst2_hbm.at[wblk], dstblk)

        front(0, 0, False)
        front(1, 1, False)
        front(2, 2, False)
        back(0, 0)
        front(3, 3, False)
        back(1, 1)

        def grp(kk, _):
            k0 = _GRP0 + kk * NBUF
            for b4 in range(NBUF):
                k = k0 + b4
                b = (_GRP0 + b4) % NBUF
                front(k + 2, (b + 2) % NBUF, True)
                back(k, b)
            return 0
        lax.fori_loop(0, _GRPN, grp, 0)

        for k in range(_EPI0, CH_BLK):
            if k + 2 < CH_BLK:
                front(k + 2, (k + 2) % NBUF, True)
            back(k, k % NBUF)
        for k in range(CH_BLK - NBUF, CH_BLK):
            b = k % NBUF
            pltpu.make_async_copy(rows[b], acc.at[dstblk.at[0]],
                                  ssem[b]).wait()
    plsc.subcore_barrier()

    pltpu.sync_copy(acc.at[pl.ds(sid * RPS, RPS)],
                    out_hbm.at[pl.ds(out_row0 + sid * RPS, RPS)])
    plsc.subcore_barrier()


_SC_SCRATCH = (
    [pltpu.VMEM_SHARED((N_ACC, D_IN), jnp.float32)]
    + [pltpu.VMEM((CH_BLK, C), jnp.int32)]
    + [pltpu.VMEM((CH_BLK, C), jnp.int32)]
    + [pltpu.VMEM((L, D_IN), jnp.float32)]
    + [pltpu.VMEM((C, D_IN), jnp.float32)] * NBUF
    + [pltpu.SemaphoreType.DMA] * (2 * NBUF)
)


def _unpack_scratch(scratch):
    acc, srcblk, dstblk, zrows = scratch[:4]
    rows = scratch[4:4 + NBUF]
    gsem = scratch[4 + NBUF:4 + 2 * NBUF]
    ssem = scratch[4 + 2 * NBUF:4 + 3 * NBUF]
    return acc, srcblk, dstblk, zrows, rows, gsem, ssem


@functools.partial(
    pl.kernel,
    out_type=jax.ShapeDtypeStruct((NC * N_ACC, D_IN), jnp.float32),
    mesh=_mesh, scratch_types=_SC_SCRATCH,
)
def _sc_scatter_x(src2_hbm, dst2_hbm, tab_hbm, out_hbm, *scratch):
    cid = lax.axis_index("c")
    sid = lax.axis_index("s")
    wid = sid * NC + cid
    acc, srcblk, dstblk, zrows, rows, gsem, ssem = _unpack_scratch(scratch)
    _scatter_pass(src2_hbm, dst2_hbm, tab_hbm, out_hbm, cid * N_ACC,
                  acc, srcblk, dstblk, zrows, rows, gsem, ssem, sid, wid)


@functools.partial(
    pl.kernel,
    out_type=jax.ShapeDtypeStruct((NC * 2 * N_ACC, D_IN), jnp.float32),
    mesh=_mesh, scratch_types=_SC_SCRATCH,
)
def _sc_scatter_h(src2_hbm, dst2_hbm, ta_hbm, tb_hbm, out_hbm, *scratch):
    cid = lax.axis_index("c")
    sid = lax.axis_index("s")
    wid = sid * NC + cid
    acc, srcblk, dstblk, zrows, rows, gsem, ssem = _unpack_scratch(scratch)
    for half, tab in enumerate((ta_hbm, tb_hbm)):
        _scatter_pass(src2_hbm, dst2_hbm, tab, out_hbm,
                      (cid * 2 + half) * N_ACC,
                      acc, srcblk, dstblk, zrows, rows, gsem, ssem, sid, wid)


M_BLK = 400
GRID_M = N // M_BLK


def _tc1_body(gpa, gpb, dv, w1, b1, hpa, hpb):
    d = dv[...]
    g0 = (gpa[...] + gpb[...]) * d
    h = jnp.dot(g0, w1[...], preferred_element_type=jnp.float32) + b1[...]
    h = jnp.maximum(h, 0.0) * d
    hpa[...] = h[:, :D_IN]
    hpb[...] = h[:, D_IN:]


_tc1 = pl.pallas_call(
    _tc1_body,
    grid=(GRID_M,),
    in_specs=[
        pl.BlockSpec((M_BLK, D_IN), lambda i: (i, 0)),
        pl.BlockSpec((M_BLK, D_IN), lambda i: (i, 0)),
        pl.BlockSpec((M_BLK, 1), lambda i: (i, 0)),
        pl.BlockSpec((D_IN, D_HID), lambda i: (0, 0)),
        pl.BlockSpec((1, D_HID), lambda i: (0, 0)),
    ],
    out_specs=[pl.BlockSpec((M_BLK, D_IN), lambda i: (i, 0))] * 2,
    out_shape=[jax.ShapeDtypeStruct((N, D_IN), jnp.float32)] * 2,
)


def _tc2_body(a0, a1, b0, b1, dv, wma, wmb, wla, wlb, bm, bl, mu, ls):
    d = dv[...]
    ga = (a0[...] + a1[...]) * d
    gb = (b0[...] + b1[...]) * d
    mu[...] = (jnp.dot(ga, wma[...], preferred_element_type=jnp.float32)
               + jnp.dot(gb, wmb[...], preferred_element_type=jnp.float32)
               + bm[...])
    ls[...] = (jnp.dot(ga, wla[...], preferred_element_type=jnp.float32)
               + jnp.dot(gb, wlb[...], preferred_element_type=jnp.float32)
               + bl[...])


_tc2 = pl.pallas_call(
    _tc2_body,
    grid=(GRID_M,),
    in_specs=(
        [pl.BlockSpec((M_BLK, D_IN), lambda i: (i, 0))] * 4
        + [pl.BlockSpec((M_BLK, 1), lambda i: (i, 0))]
        + [pl.BlockSpec((D_IN, D_IN), lambda i: (0, 0))] * 4
        + [pl.BlockSpec((1, D_IN), lambda i: (0, 0))] * 2
    ),
    out_specs=[pl.BlockSpec((M_BLK, D_IN), lambda i: (i, 0))] * 2,
    out_shape=[jax.ShapeDtypeStruct((N, D_IN), jnp.float32)] * 2,
)


def kernel(x, edge_index, W1, b1, Wmu, bmu, Wls, bls):
    ei = edge_index.astype(jnp.int32)
    loop = jnp.arange(N, dtype=jnp.int32)
    src = jnp.concatenate(
        [ei[0], loop, jnp.zeros((E_PAD - E_RAW,), jnp.int32)])
    dst = jnp.concatenate(
        [ei[1], loop, jnp.full((E_PAD - E_RAW,), N, jnp.int32)])
    src2 = src.reshape(NW * (CH_PER_W // CH_BLK), CH_BLK, C)
    dst2 = dst.reshape(NW * (CH_PER_W // CH_BLK), CH_BLK, C)

    dinv, xp = _sc_prep(dst, x)
    dv = dinv[:N, None]

    s0 = _sc_scatter_x(src2, dst2, xp)
    hpa, hpb = _tc1(s0[:N], s0[N_ACC:N_ACC + N], dv, W1, b1.reshape(1, -1))

    s1 = _sc_scatter_h(src2, dst2, hpa, hpb)
    a0 = s1[:N]
    b0 = s1[N_ACC:N_ACC + N]
    a1 = s1[2 * N_ACC:2 * N_ACC + N]
    b1h = s1[3 * N_ACC:3 * N_ACC + N]

    mu, ls = _tc2(a0, a1, b0, b1h, dv,
                  Wmu[:D_IN], Wmu[D_IN:], Wls[:D_IN], Wls[D_IN:],
                  bmu.reshape(1, -1), bls.reshape(1, -1))
    return (mu, ls)


# speedup vs baseline: 18.7963x; 1.1158x over previous
import functools

import jax
import jax.numpy as jnp
from jax import lax
from jax.experimental import pallas as pl
from jax.experimental.pallas import tpu as pltpu
from jax.experimental.pallas import tpu_sc as plsc

N = 10000
D_IN = 128
D_HID = 256
NC, NS, L = 2, 16, 16
NW = NC * NS
N_ACC = 10240
RPS = N_ACC // NS
C = 64
CDEG = 128
E_RAW = 320000 + N
CH_PER_W = -(-E_RAW // (NW * C))
CH_BLK = CH_PER_W // 6
E_PAD = NW * C * CH_PER_W
EPW = C * CH_PER_W
NBLK = 6
BPS = NW * NBLK // NS
X_CH = N // L

_mesh = plsc.VectorSubcoreMesh(
    core_axis_name="c", subcore_axis_name="s", num_cores=NC, num_subcores=NS)


@functools.partial(
    pl.kernel,
    out_type=(jax.ShapeDtypeStruct((N_ACC,), jnp.float32),
              jax.ShapeDtypeStruct((N, D_IN), jnp.float32)),
    mesh=_mesh,
    scratch_types=[
        pltpu.VMEM_SHARED((N_ACC,), jnp.float32),
        pltpu.VMEM_SHARED((N_ACC,), jnp.float32),
        pltpu.VMEM((RPS,), jnp.float32),
        pltpu.VMEM((C,), jnp.float32),
        pltpu.VMEM((CH_BLK, C), jnp.int32),
        pltpu.VMEM((CH_BLK, C), jnp.int32),
        pltpu.VMEM((N_ACC,), jnp.float32),
        pltpu.VMEM((L, D_IN), jnp.float32),
        pltpu.VMEM((L, D_IN), jnp.float32),
        pltpu.SemaphoreType.DMA,
        pltpu.SemaphoreType.DMA,
    ],
)
def _sc_prep(dst2_hbm, x_hbm, dinv_out, xp_out,
             deg_sh, dinv_sh, work, ones, dblk0, dblk1, dinv_full,
             xbuf, xpbuf, dsem0, dsem1):
    cid = lax.axis_index("c")
    sid = lax.axis_index("s")

    def fill_zero(i, _):
        work[pl.ds(i * L, L)] = jnp.zeros((L,), jnp.float32)
        return 0
    lax.fori_loop(0, RPS // L, fill_zero, 0)

    def fill_one(i, _):
        ones[pl.ds(i * L, L)] = jnp.ones((L,), jnp.float32)
        return 0
    lax.fori_loop(0, C // L, fill_one, 0)

    pltpu.sync_copy(work, deg_sh.at[pl.ds(sid * RPS, RPS)])
    plsc.subcore_barrier()

    dblk = (dblk0, dblk1)
    dsem = (dsem0, dsem1)
    bbase = sid * BPS

    def dload(j, b):
        pltpu.sync_copy(dst2_hbm.at[bbase + j], dblk[b])

    def dissue(b):
        def body(r, _):
            pltpu.async_copy(ones, deg_sh.at[dblk[b].at[r]],
                             dsem[b], add=True)
            return 0
        lax.fori_loop(0, CH_BLK, body, 0)

    def ddrain(b):
        def body(r, _):
            pltpu.make_async_copy(ones, deg_sh.at[dblk[b].at[0]],
                                  dsem[b]).wait()
            return 0
        lax.fori_loop(0, CH_BLK, body, 0)

    dload(0, 0)
    dissue(0)
    dload(1, 1)
    dissue(1)

    def dgrp(jj, _):
        for b in range(2):
            j = 2 + 2 * jj + b
            ddrain(b)
            dload(j, b)
            dissue(b)
        return 0
    lax.fori_loop(0, (BPS - 2) // 2, dgrp, 0)
    for b in range(2):
        ddrain(b)
    plsc.subcore_barrier()

    pltpu.sync_copy(deg_sh.at[pl.ds(sid * RPS, RPS)], work)

    def rsq(i, _):
        d = jnp.maximum(work[pl.ds(i * L, L)], 1.0)
        bits = lax.bitcast_convert_type(d, jnp.int32)
        bits = jnp.int32(0x5F3759DF) - lax.shift_right_logical(bits, 1)
        y = lax.bitcast_convert_type(bits, jnp.float32)
        y = y * (1.5 - 0.5 * d * y * y)
        y = y * (1.5 - 0.5 * d * y * y)
        y = y * (1.5 - 0.5 * d * y * y)
        work[pl.ds(i * L, L)] = y
        return 0
    lax.fori_loop(0, RPS // L, rsq, 0)

    pltpu.sync_copy(work, dinv_sh.at[pl.ds(sid * RPS, RPS)])

    @pl.when(cid == 0)
    def _():
        pltpu.sync_copy(work, dinv_out.at[pl.ds(sid * RPS, RPS)])

    plsc.subcore_barrier()
    pltpu.sync_copy(dinv_sh, dinv_full)

    wid = sid * NC + cid

    def x_step(k, _):
        t = k * NW + wid

        @pl.when(t < X_CH)
        def _():
            r0 = t * L
            pltpu.sync_copy(x_hbm.at[pl.ds(r0, L)], xbuf)
            dvec = dinv_full[pl.ds(r0, L)]
            for j in range(L):
                dv = dvec[j]
                for v in range(D_IN // L):
                    xpbuf[j, pl.ds(v * L, L)] = xbuf[j, pl.ds(v * L, L)] * dv
            pltpu.sync_copy(xpbuf, xp_out.at[pl.ds(r0, L)])
        return 0
    lax.fori_loop(0, -(-X_CH // NW), x_step, 0)


NBUF = 4
_GRP0 = 2
_GRPN = (CH_BLK - 1 - 2 - _GRP0) // NBUF
_EPI0 = _GRP0 + _GRPN * NBUF


def _scatter_pass(src2_hbm, dst2_hbm, tab_hbm, out_hbm, out_row0,
                  acc, srcblk, dstblks, zrows, rows, gsem, ssem, sid, wid):
    def zfill(i, _):
        for v in range(D_IN // L):
            zrows[i, pl.ds(v * L, L)] = jnp.zeros((L,), jnp.float32)
        return 0
    lax.fori_loop(0, L, zfill, 0)

    def zslice(i, _):
        pltpu.sync_copy(zrows, acc.at[pl.ds(sid * RPS + i * L, L)])
        return 0
    lax.fori_loop(0, RPS // L, zslice, 0)
    plsc.subcore_barrier()

    for blk in range(NBLK):
        dstblk = dstblks[blk % 2]
        wblk = wid * NBLK + blk
        pltpu.sync_copy(src2_hbm.at[wblk], srcblk)
        pltpu.sync_copy(dst2_hbm.at[wblk], dstblk)

        def front(k, b, wait_s):
            if wait_s:
                pltpu.make_async_copy(rows[b], acc.at[dstblk.at[0]],
                                      ssem[b]).wait()
            pltpu.async_copy(tab_hbm.at[srcblk.at[k]], rows[b], gsem[b])

        def back(k, b):
            pltpu.make_async_copy(tab_hbm.at[srcblk.at[k]], rows[b],
                                  gsem[b]).wait()
            pltpu.async_copy(rows[b], acc.at[dstblk.at[k]],
                             ssem[b], add=True)

        front(0, 0, blk > 0)
        front(1, 1, blk > 0)
        front(2, 2, blk > 0)
        back(0, 0)
        front(3, 3, blk > 0)
        back(1, 1)

        def grp(kk, _):
            k0 = _GRP0 + kk * NBUF
            for b4 in range(NBUF):
                k = k0 + b4
                b = (_GRP0 + b4) % NBUF
                front(k + 2, (b + 2) % NBUF, True)
                back(k, b)
            return 0
        lax.fori_loop(0, _GRPN, grp, 0)

        for k in range(_EPI0, CH_BLK):
            if k + 2 < CH_BLK:
                front(k + 2, (k + 2) % NBUF, True)
            back(k, k % NBUF)
        if blk == NBLK - 1:
            for k in range(CH_BLK - NBUF, CH_BLK):
                b = k % NBUF
                pltpu.make_async_copy(rows[b], acc.at[dstblk.at[0]],
                                      ssem[b]).wait()
    plsc.subcore_barrier()

    pltpu.sync_copy(acc.at[pl.ds(sid * RPS, RPS)],
                    out_hbm.at[pl.ds(out_row0 + sid * RPS, RPS)])
    plsc.subcore_barrier()


_SC_SCRATCH = (
    [pltpu.VMEM_SHARED((N_ACC, D_IN), jnp.float32)]
    + [pltpu.VMEM((CH_BLK, C), jnp.int32)]
    + [pltpu.VMEM((CH_BLK, C), jnp.int32)] * 2
    + [pltpu.VMEM((L, D_IN), jnp.float32)]
    + [pltpu.VMEM((C, D_IN), jnp.float32)] * NBUF
    + [pltpu.SemaphoreType.DMA] * (2 * NBUF)
)


def _unpack_scratch(scratch):
    acc, srcblk, dstblk0, dstblk1, zrows = scratch[:5]
    rows = scratch[5:5 + NBUF]
    gsem = scratch[5 + NBUF:5 + 2 * NBUF]
    ssem = scratch[5 + 2 * NBUF:5 + 3 * NBUF]
    return acc, srcblk, (dstblk0, dstblk1), zrows, rows, gsem, ssem


@functools.partial(
    pl.kernel,
    out_type=jax.ShapeDtypeStruct((NC * N_ACC, D_IN), jnp.float32),
    mesh=_mesh, scratch_types=_SC_SCRATCH,
)
def _sc_scatter_x(src2_hbm, dst2_hbm, tab_hbm, out_hbm, *scratch):
    cid = lax.axis_index("c")
    sid = lax.axis_index("s")
    wid = sid * NC + cid
    acc, srcblk, dstblks, zrows, rows, gsem, ssem = _unpack_scratch(scratch)
    _scatter_pass(src2_hbm, dst2_hbm, tab_hbm, out_hbm, cid * N_ACC,
                  acc, srcblk, dstblks, zrows, rows, gsem, ssem, sid, wid)


@functools.partial(
    pl.kernel,
    out_type=jax.ShapeDtypeStruct((NC * 2 * N_ACC, D_IN), jnp.float32),
    mesh=_mesh, scratch_types=_SC_SCRATCH,
)
def _sc_scatter_h(src2_hbm, dst2_hbm, ta_hbm, tb_hbm, out_hbm, *scratch):
    cid = lax.axis_index("c")
    sid = lax.axis_index("s")
    wid = sid * NC + cid
    acc, srcblk, dstblks, zrows, rows, gsem, ssem = _unpack_scratch(scratch)
    for half, tab in enumerate((ta_hbm, tb_hbm)):
        _scatter_pass(src2_hbm, dst2_hbm, tab, out_hbm,
                      (cid * 2 + half) * N_ACC,
                      acc, srcblk, dstblks, zrows, rows, gsem, ssem, sid, wid)


M_BLK = 400
GRID_M = N // M_BLK


def _tc1_body(gpa, gpb, dv, w1, b1, hpa, hpb):
    d = dv[...]
    g0 = (gpa[...] + gpb[...]) * d
    h = jnp.dot(g0, w1[...], preferred_element_type=jnp.float32) + b1[...]
    h = jnp.maximum(h, 0.0) * d
    hpa[...] = h[:, :D_IN]
    hpb[...] = h[:, D_IN:]


_tc1 = pl.pallas_call(
    _tc1_body,
    grid=(GRID_M,),
    in_specs=[
        pl.BlockSpec((M_BLK, D_IN), lambda i: (i, 0)),
        pl.BlockSpec((M_BLK, D_IN), lambda i: (i, 0)),
        pl.BlockSpec((M_BLK, 1), lambda i: (i, 0)),
        pl.BlockSpec((D_IN, D_HID), lambda i: (0, 0)),
        pl.BlockSpec((1, D_HID), lambda i: (0, 0)),
    ],
    out_specs=[pl.BlockSpec((M_BLK, D_IN), lambda i: (i, 0))] * 2,
    out_shape=[jax.ShapeDtypeStruct((N, D_IN), jnp.float32)] * 2,
)


def _tc2_body(a0, a1, b0, b1, dv, wma, wmb, wla, wlb, bm, bl, mu, ls):
    d = dv[...]
    ga = (a0[...] + a1[...]) * d
    gb = (b0[...] + b1[...]) * d
    mu[...] = (jnp.dot(ga, wma[...], preferred_element_type=jnp.float32)
               + jnp.dot(gb, wmb[...], preferred_element_type=jnp.float32)
               + bm[...])
    ls[...] = (jnp.dot(ga, wla[...], preferred_element_type=jnp.float32)
               + jnp.dot(gb, wlb[...], preferred_element_type=jnp.float32)
               + bl[...])


_tc2 = pl.pallas_call(
    _tc2_body,
    grid=(GRID_M,),
    in_specs=(
        [pl.BlockSpec((M_BLK, D_IN), lambda i: (i, 0))] * 4
        + [pl.BlockSpec((M_BLK, 1), lambda i: (i, 0))]
        + [pl.BlockSpec((D_IN, D_IN), lambda i: (0, 0))] * 4
        + [pl.BlockSpec((1, D_IN), lambda i: (0, 0))] * 2
    ),
    out_specs=[pl.BlockSpec((M_BLK, D_IN), lambda i: (i, 0))] * 2,
    out_shape=[jax.ShapeDtypeStruct((N, D_IN), jnp.float32)] * 2,
)


def kernel(x, edge_index, W1, b1, Wmu, bmu, Wls, bls):
    ei = edge_index.astype(jnp.int32)
    loop = jnp.arange(N, dtype=jnp.int32)
    src = jnp.concatenate(
        [ei[0], loop, jnp.zeros((E_PAD - E_RAW,), jnp.int32)])
    dst = jnp.concatenate(
        [ei[1], loop, jnp.full((E_PAD - E_RAW,), N, jnp.int32)])
    src2 = src.reshape(NW * NBLK, CH_BLK, C)
    dst2 = dst.reshape(NW * NBLK, CH_BLK, C)

    dinv, xp = _sc_prep(dst2, x)
    dv = dinv[:N, None]

    s0 = _sc_scatter_x(src2, dst2, xp)
    hpa, hpb = _tc1(s0[:N], s0[N_ACC:N_ACC + N], dv, W1, b1.reshape(1, -1))

    s1 = _sc_scatter_h(src2, dst2, hpa, hpb)
    a0 = s1[:N]
    b0 = s1[N_ACC:N_ACC + N]
    a1 = s1[2 * N_ACC:2 * N_ACC + N]
    b1h = s1[3 * N_ACC:3 * N_ACC + N]

    mu, ls = _tc2(a0, a1, b0, b1h, dv,
                  Wmu[:D_IN], Wmu[D_IN:], Wls[:D_IN], Wls[D_IN:],
                  bmu.reshape(1, -1), bls.reshape(1, -1))
    return (mu, ls)


# speedup vs baseline: 18.8115x; 1.0008x over previous
import functools

import jax
import jax.numpy as jnp
from jax import lax
from jax.experimental import pallas as pl
from jax.experimental.pallas import tpu as pltpu
from jax.experimental.pallas import tpu_sc as plsc

N = 10000
D_IN = 128
D_HID = 256
NC, NS, L = 2, 16, 16
NW = NC * NS
N_ACC = 10240
RPS = N_ACC // NS
C = 64
CDEG = 128
E_RAW = 320000 + N
CH_PER_W = -(-E_RAW // (NW * C))
CH_BLK = CH_PER_W // 6
E_PAD = NW * C * CH_PER_W
EPW = C * CH_PER_W
NBLK = 6
BPS = NW * NBLK // NS
X_CH = N // L

_mesh = plsc.VectorSubcoreMesh(
    core_axis_name="c", subcore_axis_name="s", num_cores=NC, num_subcores=NS)


@functools.partial(
    pl.kernel,
    out_type=(jax.ShapeDtypeStruct((N_ACC,), jnp.float32),
              jax.ShapeDtypeStruct((N, D_IN), jnp.float32)),
    mesh=_mesh,
    scratch_types=[
        pltpu.VMEM_SHARED((N_ACC,), jnp.float32),
        pltpu.VMEM_SHARED((N_ACC,), jnp.float32),
        pltpu.VMEM((RPS,), jnp.float32),
        pltpu.VMEM((C,), jnp.float32),
        pltpu.VMEM((CH_BLK, C), jnp.int32),
        pltpu.VMEM((CH_BLK, C), jnp.int32),
        pltpu.VMEM((N_ACC,), jnp.float32),
        pltpu.VMEM((L, D_IN), jnp.float32),
        pltpu.VMEM((L, D_IN), jnp.float32),
        pltpu.SemaphoreType.DMA,
        pltpu.SemaphoreType.DMA,
    ],
)
def _sc_prep(dst2_hbm, x_hbm, dinv_out, xp_out,
             deg_sh, dinv_sh, work, ones, dblk0, dblk1, dinv_full,
             xbuf, xpbuf, dsem0, dsem1):
    cid = lax.axis_index("c")
    sid = lax.axis_index("s")

    def fill_zero(i, _):
        work[pl.ds(i * L, L)] = jnp.zeros((L,), jnp.float32)
        return 0
    lax.fori_loop(0, RPS // L, fill_zero, 0)

    def fill_one(i, _):
        ones[pl.ds(i * L, L)] = jnp.ones((L,), jnp.float32)
        return 0
    lax.fori_loop(0, C // L, fill_one, 0)

    pltpu.sync_copy(work, deg_sh.at[pl.ds(sid * RPS, RPS)])
    plsc.subcore_barrier()

    dblk = (dblk0, dblk1)
    dsem = (dsem0, dsem1)
    bbase = sid * BPS

    def dload(j, b):
        pltpu.sync_copy(dst2_hbm.at[bbase + j], dblk[b])

    def dissue(b):
        def body(r, _):
            pltpu.async_copy(ones, deg_sh.at[dblk[b].at[r]],
                             dsem[b], add=True)
            return 0
        lax.fori_loop(0, CH_BLK, body, 0)

    def ddrain(b):
        def body(r, _):
            pltpu.make_async_copy(ones, deg_sh.at[dblk[b].at[0]],
                                  dsem[b]).wait()
            return 0
        lax.fori_loop(0, CH_BLK, body, 0)

    dload(0, 0)
    dissue(0)
    dload(1, 1)
    dissue(1)

    def dgrp(jj, _):
        for b in range(2):
            j = 2 + 2 * jj + b
            ddrain(b)
            dload(j, b)
            dissue(b)
        return 0
    lax.fori_loop(0, (BPS - 2) // 2, dgrp, 0)
    for b in range(2):
        ddrain(b)
    plsc.subcore_barrier()

    pltpu.sync_copy(deg_sh.at[pl.ds(sid * RPS, RPS)], work)

    def rsq(i, _):
        d = jnp.maximum(work[pl.ds(i * L, L)], 1.0)
        bits = lax.bitcast_convert_type(d, jnp.int32)
        bits = jnp.int32(0x5F3759DF) - lax.shift_right_logical(bits, 1)
        y = lax.bitcast_convert_type(bits, jnp.float32)
        y = y * (1.5 - 0.5 * d * y * y)
        y = y * (1.5 - 0.5 * d * y * y)
        y = y * (1.5 - 0.5 * d * y * y)
        work[pl.ds(i * L, L)] = y
        return 0
    lax.fori_loop(0, RPS // L, rsq, 0)

    pltpu.sync_copy(work, dinv_sh.at[pl.ds(sid * RPS, RPS)])

    @pl.when(cid == 0)
    def _():
        pltpu.sync_copy(work, dinv_out.at[pl.ds(sid * RPS, RPS)])

    plsc.subcore_barrier()
    pltpu.sync_copy(dinv_sh, dinv_full)

    wid = sid * NC + cid

    def x_step(k, _):
        t = k * NW + wid

        @pl.when(t < X_CH)
        def _():
            r0 = t * L
            pltpu.sync_copy(x_hbm.at[pl.ds(r0, L)], xbuf)
            dvec = dinv_full[pl.ds(r0, L)]
            for j in range(L):
                dv = dvec[j]
                for v in range(D_IN // L):
                    xpbuf[j, pl.ds(v * L, L)] = xbuf[j, pl.ds(v * L, L)] * dv
            pltpu.sync_copy(xpbuf, xp_out.at[pl.ds(r0, L)])
        return 0
    lax.fori_loop(0, -(-X_CH // NW), x_step, 0)


NBUF = 4
_GRP0 = 2
_GRPN = (CH_BLK - 1 - 2 - _GRP0) // NBUF
_EPI0 = _GRP0 + _GRPN * NBUF


def _scatter_pass(src2_hbm, dst2_hbm, tab_hbm, out_hbm, out_row0,
                  acc, srcblk, dstblks, zrows, rows, gsem, ssem, sid, wid):
    def zfill(i, _):
        for v in range(D_IN // L):
            zrows[i, pl.ds(v * L, L)] = jnp.zeros((L,), jnp.float32)
        return 0
    lax.fori_loop(0, L, zfill, 0)

    def zslice(i, _):
        pltpu.async_copy(zrows, acc.at[pl.ds(sid * RPS + i * L, L)],
                         ssem[0])
        return 0
    lax.fori_loop(0, RPS // L, zslice, 0)

    def zdrain(i, _):
        pltpu.make_async_copy(zrows, acc.at[pl.ds(sid * RPS, L)],
                              ssem[0]).wait()
        return 0
    lax.fori_loop(0, RPS // L, zdrain, 0)
    plsc.subcore_barrier()

    for blk in range(NBLK):
        dstblk = dstblks[blk % 2]
        wblk = wid * NBLK + blk
        pltpu.sync_copy(src2_hbm.at[wblk], srcblk)
        pltpu.sync_copy(dst2_hbm.at[wblk], dstblk)

        def front(k, b, wait_s):
            if wait_s:
                pltpu.make_async_copy(rows[b], acc.at[dstblk.at[0]],
                                      ssem[b]).wait()
            pltpu.async_copy(tab_hbm.at[srcblk.at[k]], rows[b], gsem[b])

        def back(k, b):
            pltpu.make_async_copy(tab_hbm.at[srcblk.at[k]], rows[b],
                                  gsem[b]).wait()
            pltpu.async_copy(rows[b], acc.at[dstblk.at[k]],
                             ssem[b], add=True)

        front(0, 0, blk > 0)
        front(1, 1, blk > 0)
        front(2, 2, blk > 0)
        back(0, 0)
        front(3, 3, blk > 0)
        back(1, 1)

        def grp(kk, _):
            k0 = _GRP0 + kk * NBUF
            for b4 in range(NBUF):
                k = k0 + b4
                b = (_GRP0 + b4) % NBUF
                front(k + 2, (b + 2) % NBUF, True)
                back(k, b)
            return 0
        lax.fori_loop(0, _GRPN, grp, 0)

        for k in range(_EPI0, CH_BLK):
            if k + 2 < CH_BLK:
                front(k + 2, (k + 2) % NBUF, True)
            back(k, k % NBUF)
        if blk == NBLK - 1:
            for k in range(CH_BLK - NBUF, CH_BLK):
                b = k % NBUF
                pltpu.make_async_copy(rows[b], acc.at[dstblk.at[0]],
                                      ssem[b]).wait()
    plsc.subcore_barrier()

    pltpu.sync_copy(acc.at[pl.ds(sid * RPS, RPS)],
                    out_hbm.at[pl.ds(out_row0 + sid * RPS, RPS)])
    plsc.subcore_barrier()


_SC_SCRATCH = (
    [pltpu.VMEM_SHARED((N_ACC, D_IN), jnp.float32)]
    + [pltpu.VMEM((CH_BLK, C), jnp.int32)]
    + [pltpu.VMEM((CH_BLK, C), jnp.int32)] * 2
    + [pltpu.VMEM((L, D_IN), jnp.float32)]
    + [pltpu.VMEM((C, D_IN), jnp.float32)] * NBUF
    + [pltpu.SemaphoreType.DMA] * (2 * NBUF)
)


def _unpack_scratch(scratch):
    acc, srcblk, dstblk0, dstblk1, zrows = scratch[:5]
    rows = scratch[5:5 + NBUF]
    gsem = scratch[5 + NBUF:5 + 2 * NBUF]
    ssem = scratch[5 + 2 * NBUF:5 + 3 * NBUF]
    return acc, srcblk, (dstblk0, dstblk1), zrows, rows, gsem, ssem


@functools.partial(
    pl.kernel,
    out_type=jax.ShapeDtypeStruct((NC * N_ACC, D_IN), jnp.float32),
    mesh=_mesh, scratch_types=_SC_SCRATCH,
)
def _sc_scatter_x(src2_hbm, dst2_hbm, tab_hbm, out_hbm, *scratch):
    cid = lax.axis_index("c")
    sid = lax.axis_index("s")
    wid = sid * NC + cid
    acc, srcblk, dstblks, zrows, rows, gsem, ssem = _unpack_scratch(scratch)
    _scatter_pass(src2_hbm, dst2_hbm, tab_hbm, out_hbm, cid * N_ACC,
                  acc, srcblk, dstblks, zrows, rows, gsem, ssem, sid, wid)


@functools.partial(
    pl.kernel,
    out_type=jax.ShapeDtypeStruct((NC * 2 * N_ACC, D_IN), jnp.float32),
    mesh=_mesh, scratch_types=_SC_SCRATCH,
)
def _sc_scatter_h(src2_hbm, dst2_hbm, ta_hbm, tb_hbm, out_hbm, *scratch):
    cid = lax.axis_index("c")
    sid = lax.axis_index("s")
    wid = sid * NC + cid
    acc, srcblk, dstblks, zrows, rows, gsem, ssem = _unpack_scratch(scratch)
    for half, tab in enumerate((ta_hbm, tb_hbm)):
        _scatter_pass(src2_hbm, dst2_hbm, tab, out_hbm,
                      (cid * 2 + half) * N_ACC,
                      acc, srcblk, dstblks, zrows, rows, gsem, ssem, sid, wid)


M_BLK = 400
GRID_M = N // M_BLK


def _tc1_body(gpa, gpb, dv, w1, b1, hpa, hpb):
    d = dv[...]
    g0 = (gpa[...] + gpb[...]) * d
    h = jnp.dot(g0, w1[...], preferred_element_type=jnp.float32) + b1[...]
    h = jnp.maximum(h, 0.0) * d
    hpa[...] = h[:, :D_IN]
    hpb[...] = h[:, D_IN:]


_tc1 = pl.pallas_call(
    _tc1_body,
    grid=(GRID_M,),
    in_specs=[
        pl.BlockSpec((M_BLK, D_IN), lambda i: (i, 0)),
        pl.BlockSpec((M_BLK, D_IN), lambda i: (i, 0)),
        pl.BlockSpec((M_BLK, 1), lambda i: (i, 0)),
        pl.BlockSpec((D_IN, D_HID), lambda i: (0, 0)),
        pl.BlockSpec((1, D_HID), lambda i: (0, 0)),
    ],
    out_specs=[pl.BlockSpec((M_BLK, D_IN), lambda i: (i, 0))] * 2,
    out_shape=[jax.ShapeDtypeStruct((N, D_IN), jnp.float32)] * 2,
)


def _tc2_body(a0, a1, b0, b1, dv, wma, wmb, wla, wlb, bm, bl, mu, ls):
    d = dv[...]
    ga = (a0[...] + a1[...]) * d
    gb = (b0[...] + b1[...]) * d
    mu[...] = (jnp.dot(ga, wma[...], preferred_element_type=jnp.float32)
               + jnp.dot(gb, wmb[...], preferred_element_type=jnp.float32)
               + bm[...])
    ls[...] = (jnp.dot(ga, wla[...], preferred_element_type=jnp.float32)
               + jnp.dot(gb, wlb[...], preferred_element_type=jnp.float32)
               + bl[...])


_tc2 = pl.pallas_call(
    _tc2_body,
    grid=(GRID_M,),
    in_specs=(
        [pl.BlockSpec((M_BLK, D_IN), lambda i: (i, 0))] * 4
        + [pl.BlockSpec((M_BLK, 1), lambda i: (i, 0))]
        + [pl.BlockSpec((D_IN, D_IN), lambda i: (0, 0))] * 4
        + [pl.BlockSpec((1, D_IN), lambda i: (0, 0))] * 2
    ),
    out_specs=[pl.BlockSpec((M_BLK, D_IN), lambda i: (i, 0))] * 2,
    out_shape=[jax.ShapeDtypeStruct((N, D_IN), jnp.float32)] * 2,
)


def kernel(x, edge_index, W1, b1, Wmu, bmu, Wls, bls):
    ei = edge_index.astype(jnp.int32)
    loop = jnp.arange(N, dtype=jnp.int32)
    src = jnp.concatenate(
        [ei[0], loop, jnp.zeros((E_PAD - E_RAW,), jnp.int32)])
    dst = jnp.concatenate(
        [ei[1], loop, jnp.full((E_PAD - E_RAW,), N, jnp.int32)])
    src2 = src.reshape(NW * NBLK, CH_BLK, C)
    dst2 = dst.reshape(NW * NBLK, CH_BLK, C)

    dinv, xp = _sc_prep(dst2, x)
    dv = dinv[:N, None]

    s0 = _sc_scatter_x(src2, dst2, xp)
    hpa, hpb = _tc1(s0[:N], s0[N_ACC:N_ACC + N], dv, W1, b1.reshape(1, -1))

    s1 = _sc_scatter_h(src2, dst2, hpa, hpb)
    a0 = s1[:N]
    b0 = s1[N_ACC:N_ACC + N]
    a1 = s1[2 * N_ACC:2 * N_ACC + N]
    b1h = s1[3 * N_ACC:3 * N_ACC + N]

    mu, ls = _tc2(a0, a1, b0, b1h, dv,
                  Wmu[:D_IN], Wmu[D_IN:], Wls[:D_IN], Wls[D_IN:],
                  bmu.reshape(1, -1), bls.reshape(1, -1))
    return (mu, ls)


# speedup vs baseline: 18.9692x; 1.0084x over previous
import functools

import jax
import jax.numpy as jnp
from jax import lax
from jax.experimental import pallas as pl
from jax.experimental.pallas import tpu as pltpu
from jax.experimental.pallas import tpu_sc as plsc

N = 10000
D_IN = 128
D_HID = 256
NC, NS, L = 2, 16, 16
NW = NC * NS
N_ACC = 10240
RPS = N_ACC // NS
C = 64
E_RAW = 320000 + N
CH_PER_W = -(-E_RAW // (NW * C))
CH_BLK = CH_PER_W // 6
E_PAD = NW * C * CH_PER_W
EPW = C * CH_PER_W
NBLK = 6
BPS = NW * NBLK // NS
X_CH = N // L

_mesh = plsc.VectorSubcoreMesh(
    core_axis_name="c", subcore_axis_name="s", num_cores=NC, num_subcores=NS)


@functools.partial(
    pl.kernel,
    out_type=(jax.ShapeDtypeStruct((N_ACC,), jnp.float32),
              jax.ShapeDtypeStruct((N, D_IN), jnp.float32)),
    mesh=_mesh,
    scratch_types=[
        pltpu.VMEM_SHARED((N_ACC,), jnp.float32),
        pltpu.VMEM_SHARED((N_ACC,), jnp.float32),
        pltpu.VMEM((RPS,), jnp.float32),
        pltpu.VMEM((C,), jnp.float32),
        pltpu.VMEM((CH_BLK, C), jnp.int32),
        pltpu.VMEM((CH_BLK, C), jnp.int32),
        pltpu.VMEM((N_ACC,), jnp.float32),
        pltpu.VMEM((L, D_IN), jnp.float32),
        pltpu.VMEM((L, D_IN), jnp.float32),
        pltpu.SemaphoreType.DMA,
        pltpu.SemaphoreType.DMA,
    ],
)
def _sc_prep(dst2_hbm, x_hbm, dinv_out, xp_out,
             deg_sh, dinv_sh, work, ones, dblk0, dblk1, dinv_full,
             xbuf, xpbuf, dsem0, dsem1):
    cid = lax.axis_index("c")
    sid = lax.axis_index("s")

    def fill_zero(i, _):
        work[pl.ds(i * L, L)] = jnp.zeros((L,), jnp.float32)
        return 0
    lax.fori_loop(0, RPS // L, fill_zero, 0)

    def fill_one(i, _):
        ones[pl.ds(i * L, L)] = jnp.ones((L,), jnp.float32)
        return 0
    lax.fori_loop(0, C // L, fill_one, 0)

    pltpu.sync_copy(work, deg_sh.at[pl.ds(sid * RPS, RPS)])
    plsc.subcore_barrier()

    dblk = (dblk0, dblk1)
    dsem = (dsem0, dsem1)
    bbase = sid * BPS

    def dload(j, b):
        pltpu.sync_copy(dst2_hbm.at[bbase + j], dblk[b])

    def dissue(b):
        def body(r, _):
            pltpu.async_copy(ones, deg_sh.at[dblk[b].at[r]],
                             dsem[b], add=True)
            return 0
        lax.fori_loop(0, CH_BLK, body, 0)

    def ddrain(b):
        def body(r, _):
            pltpu.make_async_copy(ones, deg_sh.at[dblk[b].at[0]],
                                  dsem[b]).wait()
            return 0
        lax.fori_loop(0, CH_BLK, body, 0)

    dload(0, 0)
    dissue(0)
    dload(1, 1)
    dissue(1)

    def dgrp(jj, _):
        for b in range(2):
            j = 2 + 2 * jj + b
            ddrain(b)
            dload(j, b)
            dissue(b)
        return 0
    lax.fori_loop(0, (BPS - 2) // 2, dgrp, 0)
    for b in range(2):
        ddrain(b)
    plsc.subcore_barrier()

    pltpu.sync_copy(deg_sh.at[pl.ds(sid * RPS, RPS)], work)

    def rsq(i, _):
        d = jnp.maximum(work[pl.ds(i * L, L)], 1.0)
        bits = lax.bitcast_convert_type(d, jnp.int32)
        bits = jnp.int32(0x5F3759DF) - lax.shift_right_logical(bits, 1)
        y = lax.bitcast_convert_type(bits, jnp.float32)
        y = y * (1.5 - 0.5 * d * y * y)
        y = y * (1.5 - 0.5 * d * y * y)
        y = y * (1.5 - 0.5 * d * y * y)
        work[pl.ds(i * L, L)] = y
        return 0
    lax.fori_loop(0, RPS // L, rsq, 0)

    pltpu.sync_copy(work, dinv_sh.at[pl.ds(sid * RPS, RPS)])

    @pl.when(cid == 0)
    def _():
        pltpu.sync_copy(work, dinv_out.at[pl.ds(sid * RPS, RPS)])

    plsc.subcore_barrier()
    pltpu.sync_copy(dinv_sh, dinv_full)

    wid = sid * NC + cid

    def x_step(k, _):
        t = k * NW + wid

        @pl.when(t < X_CH)
        def _():
            r0 = t * L
            pltpu.sync_copy(x_hbm.at[pl.ds(r0, L)], xbuf)
            dvec = dinv_full[pl.ds(r0, L)]
            for j in range(L):
                dv = dvec[j]
                for v in range(D_IN // L):
                    xpbuf[j, pl.ds(v * L, L)] = xbuf[j, pl.ds(v * L, L)] * dv
            pltpu.sync_copy(xpbuf, xp_out.at[pl.ds(r0, L)])
        return 0
    lax.fori_loop(0, -(-X_CH // NW), x_step, 0)


NBUF = 4
_GRP0 = 2
_GRPN = (CH_BLK - 1 - 2 - _GRP0) // NBUF
_EPI0 = _GRP0 + _GRPN * NBUF


def _scatter_pass(src2_hbm, dst2_hbm, tab_hbm, out_hbm, out_row0,
                  acc, srcblk, dstblks, zrows, rows, gsem, ssem, sid, wid):
    def zfill(i, _):
        for v in range(D_IN // L):
            zrows[i, pl.ds(v * L, L)] = jnp.zeros((L,), jnp.float32)
        return 0
    lax.fori_loop(0, L, zfill, 0)

    def zslice(i, _):
        pltpu.async_copy(zrows, acc.at[pl.ds(sid * RPS + i * L, L)],
                         ssem[0])
        return 0
    lax.fori_loop(0, RPS // L, zslice, 0)

    def zdrain(i, _):
        pltpu.make_async_copy(zrows, acc.at[pl.ds(sid * RPS, L)],
                              ssem[0]).wait()
        return 0
    lax.fori_loop(0, RPS // L, zdrain, 0)
    plsc.subcore_barrier()

    for blk in range(NBLK):
        dstblk = dstblks[blk % 2]
        wblk = wid * NBLK + blk
        pltpu.sync_copy(src2_hbm.at[wblk], srcblk)
        pltpu.sync_copy(dst2_hbm.at[wblk], dstblk)

        def front(k, b, wait_s):
            if wait_s:
                pltpu.make_async_copy(rows[b], acc.at[dstblk.at[0]],
                                      ssem[b]).wait()
            pltpu.async_copy(tab_hbm.at[srcblk.at[k]], rows[b], gsem[b])

        def back(k, b):
            pltpu.make_async_copy(tab_hbm.at[srcblk.at[k]], rows[b],
                                  gsem[b]).wait()
            pltpu.async_copy(rows[b], acc.at[dstblk.at[k]],
                             ssem[b], add=True)

        front(0, 0, blk > 0)
        front(1, 1, blk > 0)
        front(2, 2, blk > 0)
        back(0, 0)
        front(3, 3, blk > 0)
        back(1, 1)

        def grp(kk, _):
            k0 = _GRP0 + kk * NBUF
            for b4 in range(NBUF):
                k = k0 + b4
                b = (_GRP0 + b4) % NBUF
                front(k + 2, (b + 2) % NBUF, True)
                back(k, b)
            return 0
        lax.fori_loop(0, _GRPN, grp, 0)

        for k in range(_EPI0, CH_BLK):
            if k + 2 < CH_BLK:
                front(k + 2, (k + 2) % NBUF, True)
            back(k, k % NBUF)
        if blk == NBLK - 1:
            for k in range(CH_BLK - NBUF, CH_BLK):
                b = k % NBUF
                pltpu.make_async_copy(rows[b], acc.at[dstblk.at[0]],
                                      ssem[b]).wait()
    plsc.subcore_barrier()

    pltpu.sync_copy(acc.at[pl.ds(sid * RPS, RPS)],
                    out_hbm.at[pl.ds(out_row0 + sid * RPS, RPS)])
    plsc.subcore_barrier()


_SC_SCRATCH = (
    [pltpu.VMEM_SHARED((N_ACC, D_IN), jnp.float32)]
    + [pltpu.VMEM((CH_BLK, C), jnp.int32)]
    + [pltpu.VMEM((CH_BLK, C), jnp.int32)] * 2
    + [pltpu.VMEM((L, D_IN), jnp.float32)]
    + [pltpu.VMEM((C, D_IN), jnp.float32)] * NBUF
    + [pltpu.SemaphoreType.DMA] * (2 * NBUF)
)


def _unpack_scratch(scratch):
    acc, srcblk, dstblk0, dstblk1, zrows = scratch[:5]
    rows = scratch[5:5 + NBUF]
    gsem = scratch[5 + NBUF:5 + 2 * NBUF]
    ssem = scratch[5 + 2 * NBUF:5 + 3 * NBUF]
    return acc, srcblk, (dstblk0, dstblk1), zrows, rows, gsem, ssem


@functools.partial(
    pl.kernel,
    out_type=jax.ShapeDtypeStruct((NC * N_ACC, D_IN), jnp.float32),
    mesh=_mesh, scratch_types=_SC_SCRATCH,
)
def _sc_scatter_x(src2_hbm, dst2_hbm, tab_hbm, out_hbm, *scratch):
    cid = lax.axis_index("c")
    sid = lax.axis_index("s")
    wid = sid * NC + cid
    acc, srcblk, dstblks, zrows, rows, gsem, ssem = _unpack_scratch(scratch)
    _scatter_pass(src2_hbm, dst2_hbm, tab_hbm, out_hbm, cid * N_ACC,
                  acc, srcblk, dstblks, zrows, rows, gsem, ssem, sid, wid)


@functools.partial(
    pl.kernel,
    out_type=jax.ShapeDtypeStruct((NC * 2 * N_ACC, D_IN), jnp.float32),
    mesh=_mesh, scratch_types=_SC_SCRATCH,
)
def _sc_scatter_h(src2_hbm, dst2_hbm, ta_hbm, tb_hbm, out_hbm, *scratch):
    cid = lax.axis_index("c")
    sid = lax.axis_index("s")
    wid = sid * NC + cid
    acc, srcblk, dstblks, zrows, rows, gsem, ssem = _unpack_scratch(scratch)
    for half, tab in enumerate((ta_hbm, tb_hbm)):
        _scatter_pass(src2_hbm, dst2_hbm, tab, out_hbm,
                      (cid * 2 + half) * N_ACC,
                      acc, srcblk, dstblks, zrows, rows, gsem, ssem, sid, wid)


M_BLK = 400
GRID_M = N // M_BLK


def _tc1_body(gpa, gpb, dv, w1, b1, hpa, hpb):
    d = dv[...]
    g0 = (gpa[...] + gpb[...]) * d
    h = jnp.dot(g0, w1[...], preferred_element_type=jnp.float32) + b1[...]
    h = jnp.maximum(h, 0.0) * d
    hpa[...] = h[:, :D_IN]
    hpb[...] = h[:, D_IN:]


_tc1 = pl.pallas_call(
    _tc1_body,
    grid=(GRID_M,),
    in_specs=[
        pl.BlockSpec((M_BLK, D_IN), lambda i: (i, 0)),
        pl.BlockSpec((M_BLK, D_IN), lambda i: (i, 0)),
        pl.BlockSpec((M_BLK, 1), lambda i: (i, 0)),
        pl.BlockSpec((D_IN, D_HID), lambda i: (0, 0)),
        pl.BlockSpec((1, D_HID), lambda i: (0, 0)),
    ],
    out_specs=[pl.BlockSpec((M_BLK, D_IN), lambda i: (i, 0))] * 2,
    out_shape=[jax.ShapeDtypeStruct((N, D_IN), jnp.float32)] * 2,
)


def _tc2_body(a0, a1, b0, b1, dv, wma, wmb, wla, wlb, bm, bl, mu, ls):
    d = dv[...]
    ga = (a0[...] + a1[...]) * d
    gb = (b0[...] + b1[...]) * d
    mu[...] = (jnp.dot(ga, wma[...], preferred_element_type=jnp.float32)
               + jnp.dot(gb, wmb[...], preferred_element_type=jnp.float32)
               + bm[...])
    ls[...] = (jnp.dot(ga, wla[...], preferred_element_type=jnp.float32)
               + jnp.dot(gb, wlb[...], preferred_element_type=jnp.float32)
               + bl[...])


_tc2 = pl.pallas_call(
    _tc2_body,
    grid=(GRID_M,),
    in_specs=(
        [pl.BlockSpec((M_BLK, D_IN), lambda i: (i, 0))] * 4
        + [pl.BlockSpec((M_BLK, 1), lambda i: (i, 0))]
        + [pl.BlockSpec((D_IN, D_IN), lambda i: (0, 0))] * 4
        + [pl.BlockSpec((1, D_IN), lambda i: (0, 0))] * 2
    ),
    out_specs=[pl.BlockSpec((M_BLK, D_IN), lambda i: (i, 0))] * 2,
    out_shape=[jax.ShapeDtypeStruct((N, D_IN), jnp.float32)] * 2,
)


def kernel(x, edge_index, W1, b1, Wmu, bmu, Wls, bls):
    ei = edge_index.astype(jnp.int32)
    loop = jnp.arange(N, dtype=jnp.int32)
    src = jnp.concatenate(
        [ei[0], loop, jnp.zeros((E_PAD - E_RAW,), jnp.int32)])
    dst = jnp.concatenate(
        [ei[1], loop, jnp.full((E_PAD - E_RAW,), N, jnp.int32)])
    src2 = src.reshape(NW * NBLK, CH_BLK, C)
    dst2 = dst.reshape(NW * NBLK, CH_BLK, C)

    dinv, xp = _sc_prep(dst2, x)
    dv = dinv[:N, None]

    s0 = _sc_scatter_x(src2, dst2, xp)
    hpa, hpb = _tc1(s0[:N], s0[N_ACC:N_ACC + N], dv, W1, b1.reshape(1, -1))

    s1 = _sc_scatter_h(src2, dst2, hpa, hpb)
    a0 = s1[:N]
    b0 = s1[N_ACC:N_ACC + N]
    a1 = s1[2 * N_ACC:2 * N_ACC + N]
    b1h = s1[3 * N_ACC:3 * N_ACC + N]

    mu, ls = _tc2(a0, a1, b0, b1h, dv,
                  Wmu[:D_IN], Wmu[D_IN:], Wls[:D_IN], Wls[D_IN:],
                  bmu.reshape(1, -1), bls.reshape(1, -1))
    return (mu, ls)
